# Initial kernel scaffold; baseline (speedup 1.0000x reference)
import jax
import jax.numpy as jnp
from jax.experimental import pallas as pl


def kernel(x, edge_index, W, b):
    raise NotImplementedError("write your pallas kernel here")


# speedup vs baseline: 11.2642x; 11.2642x over previous
import dataclasses
import functools

import jax
import jax.numpy as jnp
from jax import lax
from jax.experimental import pallas as pl
from jax.experimental.pallas import tpu as pltpu
from jax.experimental.pallas import tpu_sc as plsc

N = 10000
E = 320000
D = 128
K = 5
ALPHA = 0.1
CSUM = (1.0 - ALPHA) / K

NC = 2
NS = 16
L = 16

NPAD = 10240
DH = D // NC
ROWS_T = NPAD // NS
RC = 128
NRC = ROWS_T // RC
EC = 125
ET = E // NS
NEC = ET // EC
IG = 32
NG = NEC // IG


def _sc_propagate(xs, src_a, dst_a):
  mesh = plsc.VectorSubcoreMesh(core_axis_name="c", subcore_axis_name="s")
  cp = pltpu.CompilerParams()
  if "needs_layout_passes" in pltpu.CompilerParams.__dataclass_fields__:
    cp = dataclasses.replace(cp, needs_layout_passes=False)
  if "use_tc_tiling_on_sc" in pltpu.CompilerParams.__dataclass_fields__:
    cp = dataclasses.replace(cp, use_tc_tiling_on_sc=False)

  @functools.partial(
      pl.kernel,
      compiler_params=cp,
      out_type=(
          jax.ShapeDtypeStruct((2 * NPAD, DH), jnp.float32),
          jax.ShapeDtypeStruct((2 * NPAD, DH), jnp.float32),
      ),
      mesh=mesh,
      scratch_types=[
          pltpu.VMEM_SHARED((NPAD, DH), jnp.float32),
          pltpu.VMEM((IG, EC), jnp.int32),
          pltpu.VMEM((IG, EC), jnp.int32),
          pltpu.VMEM((ROWS_T, DH), jnp.float32),
          pltpu.VMEM((RC, DH), jnp.float32),
          pltpu.VMEM((RC, DH), jnp.float32),
          pltpu.VMEM((EC, DH), jnp.float32),
          pltpu.VMEM((RC, DH), jnp.float32),
          pltpu.VMEM((ROWS_T,), jnp.float32),
      ],
  )
  def k(xs_hbm, src_hbm, dst_hbm, acc_hbm, v_hbm,
        s_sh, src_v, dst_v, acc_v, sbuf, vbuf, gbuf, zer, dis_v):
    c = lax.axis_index("c")
    t = lax.axis_index("s")
    row0 = t * ROWS_T
    grow0 = c * NPAD + row0
    src_row0 = c * (E // EC) + t * NEC
    dst_row0 = t * NEC

    @pl.loop(0, RC)
    def _(r):
      z = jnp.zeros((L,), jnp.float32)
      for f0 in range(DH // L):
        zer[r, pl.ds(f0 * L, L)] = z

    @pl.loop(0, EC)
    def _(r):
      o = jnp.full((L,), 1.0, jnp.float32)
      for f0 in range(DH // L):
        gbuf[r, pl.ds(f0 * L, L)] = o

    @pl.loop(0, ROWS_T)
    def _(r):
      z = jnp.zeros((L,), jnp.float32)
      for f0 in range(DH // L):
        acc_v[r, pl.ds(f0 * L, L)] = z

    for jj in range(NRC):
      pltpu.sync_copy(zer, s_sh.at[pl.ds(row0 + jj * RC, RC)])
    plsc.subcore_barrier()

    @pl.loop(0, NG)
    def _(g):
      pltpu.sync_copy(dst_hbm.at[pl.ds(dst_row0 + g * IG, IG)], dst_v)

      @pl.loop(0, IG)
      def _(j):
        pltpu.sync_copy(gbuf, s_sh.at[dst_v.at[j]], add=True)
    plsc.subcore_barrier()

    iota = lax.iota(jnp.int32, L)
    zcol = jnp.zeros((L,), jnp.int32)
    for jj in range(NRC):
      pltpu.sync_copy(s_sh.at[pl.ds(row0 + jj * RC, RC)], sbuf)
      pltpu.sync_copy(zer, s_sh.at[pl.ds(row0 + jj * RC, RC)])

      @pl.loop(0, RC // L)
      def _(g):
        d = plsc.load_gather(sbuf, [iota + g * L, zcol]) + 1.0
        i = plsc.bitcast(d, jnp.int32)
        y = plsc.bitcast(0x5F3759DF - (i >> 1), jnp.float32)
        for _ in range(3):
          y = y * (1.5 - 0.5 * d * y * y)
        dis_v[pl.ds(jj * RC + g * L, L)] = y

      pltpu.sync_copy(xs_hbm.at[pl.ds(grow0 + jj * RC, RC)], sbuf)

      @pl.loop(0, RC // L)
      def _(g):
        dis16 = dis_v[pl.ds(jj * RC + g * L, L)]
        for rr in range(L):
          dsc = dis16[rr]
          r = g * L + rr
          for f0 in range(DH // L):
            sl = pl.ds(f0 * L, L)
            vbuf[r, sl] = dsc * sbuf[r, sl]

      pltpu.sync_copy(vbuf, v_hbm.at[pl.ds(grow0 + jj * RC, RC)])
    plsc.subcore_barrier()

    @pl.loop(0, K)
    def _(_k):
      @pl.loop(0, NG)
      def _(g):
        pltpu.sync_copy(src_hbm.at[pl.ds(src_row0 + g * IG, IG)], src_v)
        pltpu.sync_copy(dst_hbm.at[pl.ds(dst_row0 + g * IG, IG)], dst_v)

        @pl.loop(0, IG)
        def _(j):
          pltpu.sync_copy(v_hbm.at[src_v.at[j]], gbuf)
          pltpu.sync_copy(gbuf, s_sh.at[dst_v.at[j]], add=True)

      plsc.subcore_barrier()

      for jj in range(NRC):
        pltpu.sync_copy(s_sh.at[pl.ds(row0 + jj * RC, RC)], sbuf)
        pltpu.sync_copy(zer, s_sh.at[pl.ds(row0 + jj * RC, RC)])
        pltpu.sync_copy(v_hbm.at[pl.ds(grow0 + jj * RC, RC)], vbuf)

        @pl.loop(0, RC // L)
        def _(g):
          dis16 = dis_v[pl.ds(jj * RC + g * L, L)]
          for rr in range(L):
            dsc = dis16[rr]
            d2 = dsc * dsc
            r = g * L + rr
            for f0 in range(DH // L):
              sl = pl.ds(f0 * L, L)
              tv = sbuf[r, sl] + vbuf[r, sl]
              plsc.addupdate(acc_v.at[jj * RC + r, sl], dsc * tv)
              vbuf[r, sl] = d2 * tv

        pltpu.sync_copy(vbuf, v_hbm.at[pl.ds(grow0 + jj * RC, RC)])

      plsc.subcore_barrier()

    pltpu.sync_copy(acc_v, acc_hbm.at[pl.ds(grow0, ROWS_T)])

  acc, _v = k(xs, src_a, dst_a)
  return acc


def _tc_linear(x, acc, W, b2):
  BR = 80

  def body(x_ref, a0_ref, a1_ref, w_ref, b_ref, o_ref):
    a = jnp.concatenate([a0_ref[...], a1_ref[...]], axis=1)
    y = ALPHA * x_ref[...] + CSUM * a
    o_ref[...] = lax.dot_general(
        y, w_ref[...], (((1,), (1,)), ((), ())),
        preferred_element_type=jnp.float32) + b_ref[...]

  return pl.pallas_call(
      body,
      grid=(N // BR,),
      in_specs=[
          pl.BlockSpec((BR, D), lambda i: (i, 0)),
          pl.BlockSpec((BR, DH), lambda i: (i, 0)),
          pl.BlockSpec((BR, DH), lambda i: (i + NPAD // BR, 0)),
          pl.BlockSpec((D, D), lambda i: (0, 0)),
          pl.BlockSpec((1, D), lambda i: (0, 0)),
      ],
      out_specs=pl.BlockSpec((BR, D), lambda i: (i, 0)),
      out_shape=jax.ShapeDtypeStruct((N, D), jnp.float32),
  )(x, acc, acc, W, b2)


@jax.jit
def kernel(x, edge_index, W, b):
  src = edge_index[0]
  dst = edge_index[1]
  src_a = jnp.concatenate(
      [src.reshape(E // EC, EC), (src + NPAD).reshape(E // EC, EC)], axis=0)
  dst_a = dst.reshape(E // EC, EC)
  xp = jnp.pad(x, ((0, NPAD - N), (0, 0)))
  xs = jnp.concatenate([xp[:, :DH], xp[:, DH:]], axis=0)
  acc = _sc_propagate(xs, src_a, dst_a)
  return _tc_linear(x, acc, W, b.reshape(1, D))
